# baseline (speedup 1.0000x reference)
import functools

import jax
import jax.numpy as jnp
import numpy as np
from jax import lax
from jax.experimental import pallas as pl
from jax.experimental.pallas import tpu as pltpu
from jax.experimental.pallas import tpu_sc as plsc

_NUM_FIELDS = 26
_EMBED_DIM = 16
_BATCH = 4096
_FIELD_DIM = 100000
_OFFSETS = (np.arange(_NUM_FIELDS, dtype=np.int32) * _FIELD_DIM)

_N_LOOKUPS = _BATCH * _NUM_FIELDS
_IDX_MINOR = 128
_IDX_MAJOR = _N_LOOKUPS // _IDX_MINOR

_NW = 32
_CHUNKS_PER_W = _IDX_MAJOR // _NW
_ROWS_PER_W = _N_LOOKUPS // _NW

_PAIR_INDEX = {}
_p = 0
for _i in range(_NUM_FIELDS - 1):
    for _j in range(_i + 1, _NUM_FIELDS):
        _PAIR_INDEX[(_i, _j)] = _p
        _p += 1
_N_PAIRS = _p
_IN_DIM = _NUM_FIELDS * _EMBED_DIM + _N_PAIRS
_HID = 400

_S_ROWS = 25 * 32
_w0b_src = np.zeros((_S_ROWS,), dtype=np.int32)
_w0b_valid = np.zeros((_S_ROWS, 1), dtype=np.float32)
for _j in range(1, _NUM_FIELDS):
    for _i in range(32):
        _r = (_j - 1) * 32 + _i
        if _i < _j:
            _w0b_src[_r] = _NUM_FIELDS * _EMBED_DIM + _PAIR_INDEX[(_i, _j)]
            _w0b_valid[_r, 0] = 1.0


_GRP = 4
_LK_PER_IT = _GRP * 16
_N_IT = _ROWS_PER_W // _LK_PER_IT


def _sc_gather_body(table_hbm, idx_hbm, out_hbm, ivm, rows_v, sem):
    wid = lax.axis_index("s") * 2 + lax.axis_index("c")
    pltpu.sync_copy(idx_hbm.at[wid], ivm)

    def step(i, carry):
        r0 = i >> 1
        c0 = (i & 1) * _LK_PER_IT

        def fire(q):
            vec = ivm[pl.ds(r0, 1), pl.ds(c0 + q * 16, 16)][0]
            voff = vec * _EMBED_DIM
            base = (i * _LK_PER_IT + q * 16) * _EMBED_DIM
            cps = []
            for l in range(16):
                b = pl.multiple_of(voff[l], _EMBED_DIM)
                cps.append(pltpu.async_copy(
                    table_hbm.at[pl.ds(b, _EMBED_DIM)],
                    rows_v.at[pl.ds(base + l * _EMBED_DIM, _EMBED_DIM)],
                    sem))
            return cps

        pend = {0: fire(0), 1: fire(1)}
        for q in range(_GRP):
            if q + 2 < _GRP:
                pend[q + 2] = fire(q + 2)
            for cp in pend.pop(q):
                cp.wait()
        return carry

    lax.fori_loop(0, _N_IT, step, 0)
    pltpu.sync_copy(
        rows_v, out_hbm.at[pl.ds(wid * _ROWS_PER_W * _EMBED_DIM,
                                 _ROWS_PER_W * _EMBED_DIM)])


@functools.cache
def _get_sc_gather():
    return functools.partial(
        pl.kernel,
        out_type=jax.ShapeDtypeStruct(
            (_N_LOOKUPS * _EMBED_DIM,), jnp.float32),
        mesh=plsc.VectorSubcoreMesh(core_axis_name="c", subcore_axis_name="s"),
        scratch_types=[
            pltpu.VMEM((_CHUNKS_PER_W, _IDX_MINOR), jnp.int32),
            pltpu.VMEM((_ROWS_PER_W * _EMBED_DIM,), jnp.float32),
            pltpu.SemaphoreType.DMA,
        ],
    )(_sc_gather_body)


_BT = 256


def _tc_body(emb_ref, w0a_ref, w0b_ref, s0_ref, c0_ref, w1_ref, s1_ref,
             c1_ref, w2_ref, s2_ref, c2_ref, wo_ref, bo_ref, out_ref):
    a = emb_ref[...]
    et = jnp.swapaxes(a, 0, 1)

    ki = lax.broadcasted_iota(jnp.int32, (32, _NUM_FIELDS * _EMBED_DIM), 1)
    ri = lax.broadcasted_iota(jnp.int32, (32, _NUM_FIELDS * _EMBED_DIM), 0)
    r_mat = (ki // _EMBED_DIM == ri).astype(jnp.float32)

    blocks = []
    for j in range(1, _NUM_FIELDS):
        t = et[j * _EMBED_DIM:(j + 1) * _EMBED_DIM, :]
        tiled = jnp.concatenate([t] * _NUM_FIELDS, axis=0)
        prod = et * tiled
        blocks.append(jnp.dot(r_mat, prod,
                              preferred_element_type=jnp.float32))
    s_t = jnp.concatenate(blocks, axis=0)

    dn = (((0,), (0,)), ((), ()))
    z = (lax.dot_general(et, w0a_ref[...], dn,
                         preferred_element_type=jnp.float32)
         + lax.dot_general(s_t, w0b_ref[...], dn,
                           preferred_element_type=jnp.float32))
    z = jnp.maximum(z * s0_ref[...] + c0_ref[...], 0.0)
    z = jnp.dot(z, w1_ref[...], preferred_element_type=jnp.float32)
    z = jnp.maximum(z * s1_ref[...] + c1_ref[...], 0.0)
    z = jnp.dot(z, w2_ref[...], preferred_element_type=jnp.float32)
    z = jnp.maximum(z * s2_ref[...] + c2_ref[...], 0.0)
    out_ref[...] = (jnp.dot(z, wo_ref[...],
                            preferred_element_type=jnp.float32)
                    + bo_ref[...])


def _tc_mlp(emb, w0a, w0b, s0, c0, w1, s1, c1, w2, s2, c2, wo, bo):
    full = lambda shp: pl.BlockSpec(shp, lambda i: (0, 0))
    grid = _BATCH // _BT
    return pl.pallas_call(
        _tc_body,
        grid=(grid,),
        in_specs=[
            pl.BlockSpec((_BT, _NUM_FIELDS * _EMBED_DIM), lambda i: (i, 0)),
            full(w0a.shape), full(w0b.shape), full(s0.shape), full(c0.shape),
            full(w1.shape), full(s1.shape), full(c1.shape),
            full(w2.shape), full(s2.shape), full(c2.shape),
            full(wo.shape), full(bo.shape),
        ],
        out_specs=pl.BlockSpec((_BT, 1), lambda i: (i, 0)),
        out_shape=jax.ShapeDtypeStruct((_BATCH, 1), jnp.float32),
        compiler_params=pltpu.CompilerParams(
            dimension_semantics=("arbitrary",)),
    )(emb, w0a, w0b, s0, c0, w1, s1, c1, w2, s2, c2, wo, bo)


def kernel(x, params):
    idx = (x + jnp.asarray(_OFFSETS)[None, :]).reshape(
        _NW, _CHUNKS_PER_W, _IDX_MINOR)
    rows = _get_sc_gather()(params['table'].reshape(-1), idx)
    emb = rows.reshape(_BATCH, _NUM_FIELDS * _EMBED_DIM)

    inv = 1.0 / np.sqrt(1.0 + 1e-5).astype(np.float32)
    w0 = params['W0']
    w0a = w0[:_NUM_FIELDS * _EMBED_DIM]
    w0b = jnp.take(w0, jnp.asarray(_w0b_src), axis=0) * jnp.asarray(_w0b_valid)
    scs = []
    for i in range(3):
        s = (params[f'g{i}'] * inv)[None, :]
        c = (params[f'b{i}'] * s[0] + params[f'beta{i}'])[None, :]
        scs += [s, c]
    return _tc_mlp(emb, w0a, w0b, scs[0], scs[1], params['W1'], scs[2],
                   scs[3], params['W2'], scs[4], scs[5], params['Wo'],
                   params['bo'].reshape(1, 1))


# speedup vs baseline: 1.0069x; 1.0069x over previous
import functools

import jax
import jax.numpy as jnp
import numpy as np
from jax import lax
from jax.experimental import pallas as pl
from jax.experimental.pallas import tpu as pltpu
from jax.experimental.pallas import tpu_sc as plsc

_NUM_FIELDS = 26
_EMBED_DIM = 16
_BATCH = 4096
_FIELD_DIM = 100000
_OFFSETS = (np.arange(_NUM_FIELDS, dtype=np.int32) * _FIELD_DIM)

_N_LOOKUPS = _BATCH * _NUM_FIELDS
_IDX_MINOR = 128
_IDX_MAJOR = _N_LOOKUPS // _IDX_MINOR

_NW = 32
_CHUNKS_PER_W = _IDX_MAJOR // _NW
_ROWS_PER_W = _N_LOOKUPS // _NW

_PAIR_INDEX = {}
_p = 0
for _i in range(_NUM_FIELDS - 1):
    for _j in range(_i + 1, _NUM_FIELDS):
        _PAIR_INDEX[(_i, _j)] = _p
        _p += 1
_N_PAIRS = _p
_IN_DIM = _NUM_FIELDS * _EMBED_DIM + _N_PAIRS
_HID = 400

_S_ROWS = 25 * 32
_w0b_src = np.zeros((_S_ROWS,), dtype=np.int32)
_w0b_valid = np.zeros((_S_ROWS, 1), dtype=np.float32)
for _j in range(1, _NUM_FIELDS):
    for _i in range(32):
        _r = (_j - 1) * 32 + _i
        if _i < _j:
            _w0b_src[_r] = _NUM_FIELDS * _EMBED_DIM + _PAIR_INDEX[(_i, _j)]
            _w0b_valid[_r, 0] = 1.0


_GRP = 4
_LK_PER_IT = _GRP * 16
_N_IT = _ROWS_PER_W // _LK_PER_IT


def _sc_gather_body(table_hbm, idx_hbm, out_hbm, ivm, rows_v, sem):
    wid = lax.axis_index("s") * 2 + lax.axis_index("c")
    pltpu.sync_copy(idx_hbm.at[wid], ivm)

    def step(i, carry):
        r0 = i >> 1
        c0 = (i & 1) * _LK_PER_IT

        def fire(q):
            vec = ivm[pl.ds(r0, 1), pl.ds(c0 + q * 16, 16)][0]
            va = vec >> 3
            vc = (vec & 7) * _EMBED_DIM
            rbase = (i * _LK_PER_IT + q * 16) // 8
            cps = []
            for l in range(16):
                c = pl.multiple_of(vc[l], _EMBED_DIM)
                cps.append(pltpu.async_copy(
                    table_hbm.at[pl.ds(va[l], 1), pl.ds(c, _EMBED_DIM)],
                    rows_v.at[pl.ds(rbase + l // 8, 1),
                              pl.ds((l % 8) * _EMBED_DIM, _EMBED_DIM)],
                    sem))
            return cps

        pend = {0: fire(0), 1: fire(1)}
        for q in range(_GRP):
            if q + 2 < _GRP:
                pend[q + 2] = fire(q + 2)
            for cp in pend.pop(q):
                cp.wait()
        return carry

    lax.fori_loop(0, _N_IT, step, 0)
    pltpu.sync_copy(rows_v, out_hbm.at[wid])


@functools.cache
def _get_sc_gather():
    return functools.partial(
        pl.kernel,
        out_type=jax.ShapeDtypeStruct(
            (_NW, _ROWS_PER_W // 8, 8 * _EMBED_DIM), jnp.float32),
        mesh=plsc.VectorSubcoreMesh(core_axis_name="c", subcore_axis_name="s"),
        scratch_types=[
            pltpu.VMEM((_CHUNKS_PER_W, _IDX_MINOR), jnp.int32),
            pltpu.VMEM((_ROWS_PER_W // 8, 8 * _EMBED_DIM), jnp.float32),
            pltpu.SemaphoreType.DMA,
        ],
    )(_sc_gather_body)


_DC = 8192


def _detile_body(tt_ref, out_ref):
    blk = tt_ref[...]
    out_ref[...] = jnp.swapaxes(blk, 0, 1).reshape(_DC * _EMBED_DIM)


def _tc_detile(table_t):
    v = 2600000
    grid = (v + _DC - 1) // _DC
    return pl.pallas_call(
        _detile_body,
        grid=(grid,),
        in_specs=[pl.BlockSpec((_EMBED_DIM, _DC), lambda c: (0, c))],
        out_specs=pl.BlockSpec((_DC * _EMBED_DIM,), lambda c: (c,)),
        out_shape=jax.ShapeDtypeStruct((grid * _DC * _EMBED_DIM,),
                                       jnp.float32),
        compiler_params=pltpu.CompilerParams(
            dimension_semantics=("arbitrary",)),
    )(table_t)


_BT = 256


def _tc_body(emb_ref, w0a_ref, w0b_ref, s0_ref, c0_ref, w1_ref, s1_ref,
             c1_ref, w2_ref, s2_ref, c2_ref, wo_ref, bo_ref, out_ref):
    a = emb_ref[...]
    et = jnp.swapaxes(a, 0, 1)

    ki = lax.broadcasted_iota(jnp.int32, (32, _NUM_FIELDS * _EMBED_DIM), 1)
    ri = lax.broadcasted_iota(jnp.int32, (32, _NUM_FIELDS * _EMBED_DIM), 0)
    r_mat = (ki // _EMBED_DIM == ri).astype(jnp.float32)

    blocks = []
    for j in range(1, _NUM_FIELDS):
        t = et[j * _EMBED_DIM:(j + 1) * _EMBED_DIM, :]
        tiled = jnp.concatenate([t] * _NUM_FIELDS, axis=0)
        prod = et * tiled
        blocks.append(jnp.dot(r_mat, prod,
                              preferred_element_type=jnp.float32))
    s_t = jnp.concatenate(blocks, axis=0)

    dn = (((0,), (0,)), ((), ()))
    z = (lax.dot_general(et, w0a_ref[...], dn,
                         preferred_element_type=jnp.float32)
         + lax.dot_general(s_t, w0b_ref[...], dn,
                           preferred_element_type=jnp.float32))
    z = jnp.maximum(z * s0_ref[...] + c0_ref[...], 0.0)
    z = jnp.dot(z, w1_ref[...], preferred_element_type=jnp.float32)
    z = jnp.maximum(z * s1_ref[...] + c1_ref[...], 0.0)
    z = jnp.dot(z, w2_ref[...], preferred_element_type=jnp.float32)
    z = jnp.maximum(z * s2_ref[...] + c2_ref[...], 0.0)
    out_ref[...] = (jnp.dot(z, wo_ref[...],
                            preferred_element_type=jnp.float32)
                    + bo_ref[...])


def _tc_mlp(emb, w0a, w0b, s0, c0, w1, s1, c1, w2, s2, c2, wo, bo):
    full = lambda shp: pl.BlockSpec(shp, lambda i: (0, 0))
    grid = _BATCH // _BT
    return pl.pallas_call(
        _tc_body,
        grid=(grid,),
        in_specs=[
            pl.BlockSpec((_BT, _NUM_FIELDS * _EMBED_DIM), lambda i: (i, 0)),
            full(w0a.shape), full(w0b.shape), full(s0.shape), full(c0.shape),
            full(w1.shape), full(s1.shape), full(c1.shape),
            full(w2.shape), full(s2.shape), full(c2.shape),
            full(wo.shape), full(bo.shape),
        ],
        out_specs=pl.BlockSpec((_BT, 1), lambda i: (i, 0)),
        out_shape=jax.ShapeDtypeStruct((_BATCH, 1), jnp.float32),
        compiler_params=pltpu.CompilerParams(
            dimension_semantics=("arbitrary",)),
    )(emb, w0a, w0b, s0, c0, w1, s1, c1, w2, s2, c2, wo, bo)


def kernel(x, params):
    idx = (x + jnp.asarray(_OFFSETS)[None, :]).reshape(
        _NW, _CHUNKS_PER_W, _IDX_MINOR)
    table2 = params['table'].reshape(-1, 8 * _EMBED_DIM)
    rows = _get_sc_gather()(table2, idx)
    emb = rows.reshape(_BATCH, _NUM_FIELDS * _EMBED_DIM)

    inv = 1.0 / np.sqrt(1.0 + 1e-5).astype(np.float32)
    w0 = params['W0']
    w0a = w0[:_NUM_FIELDS * _EMBED_DIM]
    w0b = jnp.take(w0, jnp.asarray(_w0b_src), axis=0) * jnp.asarray(_w0b_valid)
    scs = []
    for i in range(3):
        s = (params[f'g{i}'] * inv)[None, :]
        c = (params[f'b{i}'] * s[0] + params[f'beta{i}'])[None, :]
        scs += [s, c]
    return _tc_mlp(emb, w0a, w0b, scs[0], scs[1], params['W1'], scs[2],
                   scs[3], params['W2'], scs[4], scs[5], params['Wo'],
                   params['bo'].reshape(1, 1))


# speedup vs baseline: 2.6348x; 2.6168x over previous
import functools

import jax
import jax.numpy as jnp
import numpy as np
from jax import lax
from jax.experimental import pallas as pl
from jax.experimental.pallas import tpu as pltpu
from jax.experimental.pallas import tpu_sc as plsc

_NUM_FIELDS = 26
_EMBED_DIM = 16
_BATCH = 4096
_FIELD_DIM = 100000
_OFFSETS = (np.arange(_NUM_FIELDS, dtype=np.int32) * _FIELD_DIM)

_N_LOOKUPS = _BATCH * _NUM_FIELDS
_IDX_MINOR = 128
_IDX_MAJOR = _N_LOOKUPS // _IDX_MINOR

_NW = 32
_CHUNKS_PER_W = _IDX_MAJOR // _NW
_ROWS_PER_W = _N_LOOKUPS // _NW

_PAIR_INDEX = {}
_p = 0
for _i in range(_NUM_FIELDS - 1):
    for _j in range(_i + 1, _NUM_FIELDS):
        _PAIR_INDEX[(_i, _j)] = _p
        _p += 1
_N_PAIRS = _p
_IN_DIM = _NUM_FIELDS * _EMBED_DIM + _N_PAIRS
_HID = 400

_S_ROWS = 25 * 32
_w0b_src = np.zeros((_S_ROWS,), dtype=np.int32)
_w0b_valid = np.zeros((_S_ROWS, 1), dtype=np.float32)
for _j in range(1, _NUM_FIELDS):
    for _i in range(32):
        _r = (_j - 1) * 32 + _i
        if _i < _j:
            _w0b_src[_r] = _NUM_FIELDS * _EMBED_DIM + _PAIR_INDEX[(_i, _j)]
            _w0b_valid[_r, 0] = 1.0


_GRP = 4
_LK_PER_IT = _GRP * 16
_N_IT = _ROWS_PER_W // _LK_PER_IT


def _sc_gather_body(table_hbm, idx_hbm, out_hbm, ivm, rows_v, sem):
    wid = lax.axis_index("s") * 2 + lax.axis_index("c")
    pltpu.sync_copy(idx_hbm.at[wid], ivm)

    def step(i, carry):
        r0 = i >> 1
        c0 = (i & 1) * _LK_PER_IT

        def fire(q):
            vec = ivm[pl.ds(r0, 1), pl.ds(c0 + q * 16, 16)][0]
            va = vec >> 3
            vc = vec & 7
            rbase = (i * _LK_PER_IT + q * 16) // 8
            cps = []
            for l in range(16):
                cps.append(pltpu.async_copy(
                    table_hbm.at[va[l], vc[l]],
                    rows_v.at[rbase + l // 8,
                              pl.ds((l % 8) * _EMBED_DIM, _EMBED_DIM)],
                    sem))
            return cps

        pend = {0: fire(0), 1: fire(1)}
        for q in range(_GRP):
            if q + 2 < _GRP:
                pend[q + 2] = fire(q + 2)
            for cp in pend.pop(q):
                cp.wait()
        return carry

    lax.fori_loop(0, _N_IT, step, 0)
    pltpu.sync_copy(rows_v, out_hbm.at[wid])


@functools.cache
def _get_sc_gather():
    return functools.partial(
        pl.kernel,
        out_type=jax.ShapeDtypeStruct(
            (_NW, _ROWS_PER_W // 8, 8 * _EMBED_DIM), jnp.float32),
        mesh=plsc.VectorSubcoreMesh(core_axis_name="c", subcore_axis_name="s"),
        scratch_types=[
            pltpu.VMEM((_CHUNKS_PER_W, _IDX_MINOR), jnp.int32),
            pltpu.VMEM((_ROWS_PER_W // 8, 8 * _EMBED_DIM), jnp.float32),
            pltpu.SemaphoreType.DMA,
        ],
    )(_sc_gather_body)


_DC = 8192


def _detile_body(tt_ref, out_ref):
    blk = tt_ref[...]
    out_ref[...] = jnp.swapaxes(blk, 0, 1).reshape(_DC * _EMBED_DIM)


def _tc_detile(table_t):
    v = 2600000
    grid = (v + _DC - 1) // _DC
    return pl.pallas_call(
        _detile_body,
        grid=(grid,),
        in_specs=[pl.BlockSpec((_EMBED_DIM, _DC), lambda c: (0, c))],
        out_specs=pl.BlockSpec((_DC * _EMBED_DIM,), lambda c: (c,)),
        out_shape=jax.ShapeDtypeStruct((grid * _DC * _EMBED_DIM,),
                                       jnp.float32),
        compiler_params=pltpu.CompilerParams(
            dimension_semantics=("arbitrary",)),
    )(table_t)


_BT = 256


def _tc_body(emb_ref, w0a_ref, w0b_ref, s0_ref, c0_ref, w1_ref, s1_ref,
             c1_ref, w2_ref, s2_ref, c2_ref, wo_ref, bo_ref, out_ref):
    a = emb_ref[...]
    et = jnp.swapaxes(a, 0, 1)

    ki = lax.broadcasted_iota(jnp.int32, (32, _NUM_FIELDS * _EMBED_DIM), 1)
    ri = lax.broadcasted_iota(jnp.int32, (32, _NUM_FIELDS * _EMBED_DIM), 0)
    r_mat = (ki // _EMBED_DIM == ri).astype(jnp.float32)

    blocks = []
    for j in range(1, _NUM_FIELDS):
        t = et[j * _EMBED_DIM:(j + 1) * _EMBED_DIM, :]
        tiled = jnp.concatenate([t] * _NUM_FIELDS, axis=0)
        prod = et * tiled
        blocks.append(jnp.dot(r_mat, prod,
                              preferred_element_type=jnp.float32))
    s_t = jnp.concatenate(blocks, axis=0)

    dn = (((0,), (0,)), ((), ()))
    z = (lax.dot_general(et, w0a_ref[...], dn,
                         preferred_element_type=jnp.float32)
         + lax.dot_general(s_t, w0b_ref[...], dn,
                           preferred_element_type=jnp.float32))
    z = jnp.maximum(z * s0_ref[...] + c0_ref[...], 0.0)
    z = jnp.dot(z, w1_ref[...], preferred_element_type=jnp.float32)
    z = jnp.maximum(z * s1_ref[...] + c1_ref[...], 0.0)
    z = jnp.dot(z, w2_ref[...], preferred_element_type=jnp.float32)
    z = jnp.maximum(z * s2_ref[...] + c2_ref[...], 0.0)
    out_ref[...] = (jnp.dot(z, wo_ref[...],
                            preferred_element_type=jnp.float32)
                    + bo_ref[...])


def _tc_mlp(emb, w0a, w0b, s0, c0, w1, s1, c1, w2, s2, c2, wo, bo):
    full = lambda shp: pl.BlockSpec(shp, lambda i: (0, 0))
    grid = _BATCH // _BT
    return pl.pallas_call(
        _tc_body,
        grid=(grid,),
        in_specs=[
            pl.BlockSpec((_BT, _NUM_FIELDS * _EMBED_DIM), lambda i: (i, 0)),
            full(w0a.shape), full(w0b.shape), full(s0.shape), full(c0.shape),
            full(w1.shape), full(s1.shape), full(c1.shape),
            full(w2.shape), full(s2.shape), full(c2.shape),
            full(wo.shape), full(bo.shape),
        ],
        out_specs=pl.BlockSpec((_BT, 1), lambda i: (i, 0)),
        out_shape=jax.ShapeDtypeStruct((_BATCH, 1), jnp.float32),
        compiler_params=pltpu.CompilerParams(
            dimension_semantics=("arbitrary",)),
    )(emb, w0a, w0b, s0, c0, w1, s1, c1, w2, s2, c2, wo, bo)


def kernel(x, params):
    idx = (x + jnp.asarray(_OFFSETS)[None, :]).reshape(
        _NW, _CHUNKS_PER_W, _IDX_MINOR)
    table3 = params['table'].reshape(-1, 8, _EMBED_DIM)
    rows = _get_sc_gather()(table3, idx)
    emb = rows.reshape(_BATCH, _NUM_FIELDS * _EMBED_DIM)

    inv = 1.0 / np.sqrt(1.0 + 1e-5).astype(np.float32)
    w0 = params['W0']
    w0a = w0[:_NUM_FIELDS * _EMBED_DIM]
    w0b = jnp.take(w0, jnp.asarray(_w0b_src), axis=0) * jnp.asarray(_w0b_valid)
    scs = []
    for i in range(3):
        s = (params[f'g{i}'] * inv)[None, :]
        c = (params[f'b{i}'] * s[0] + params[f'beta{i}'])[None, :]
        scs += [s, c]
    return _tc_mlp(emb, w0a, w0b, scs[0], scs[1], params['W1'], scs[2],
                   scs[3], params['W2'], scs[4], scs[5], params['Wo'],
                   params['bo'].reshape(1, 1))


# speedup vs baseline: 2.6350x; 1.0001x over previous
import functools

import jax
import jax.numpy as jnp
import numpy as np
from jax import lax
from jax.experimental import pallas as pl
from jax.experimental.pallas import tpu as pltpu
from jax.experimental.pallas import tpu_sc as plsc

_NUM_FIELDS = 26
_EMBED_DIM = 16
_BATCH = 4096
_FIELD_DIM = 100000
_OFFSETS = (np.arange(_NUM_FIELDS, dtype=np.int32) * _FIELD_DIM)

_N_LOOKUPS = _BATCH * _NUM_FIELDS
_IDX_MINOR = 128
_IDX_MAJOR = _N_LOOKUPS // _IDX_MINOR

_NW = 32
_CHUNKS_PER_W = _IDX_MAJOR // _NW
_ROWS_PER_W = _N_LOOKUPS // _NW

_PAIR_INDEX = {}
_p = 0
for _i in range(_NUM_FIELDS - 1):
    for _j in range(_i + 1, _NUM_FIELDS):
        _PAIR_INDEX[(_i, _j)] = _p
        _p += 1
_N_PAIRS = _p
_IN_DIM = _NUM_FIELDS * _EMBED_DIM + _N_PAIRS
_HID = 400

_S_ROWS = 25 * 32
_w0b_src = np.zeros((_S_ROWS,), dtype=np.int32)
_w0b_valid = np.zeros((_S_ROWS, 1), dtype=np.float32)
for _j in range(1, _NUM_FIELDS):
    for _i in range(32):
        _r = (_j - 1) * 32 + _i
        if _i < _j:
            _w0b_src[_r] = _NUM_FIELDS * _EMBED_DIM + _PAIR_INDEX[(_i, _j)]
            _w0b_valid[_r, 0] = 1.0


_GRP = 4
_LK_PER_IT = _GRP * 16
_N_IT = _ROWS_PER_W // _LK_PER_IT


def _sc_gather_body(table_hbm, idx_hbm, out_hbm, ivm, rows_v, sem):
    wid = lax.axis_index("s") * 2 + lax.axis_index("c")
    pltpu.sync_copy(idx_hbm.at[wid], ivm)

    def step(i, carry):
        r0 = i >> 1
        c0 = (i & 1) * _LK_PER_IT

        def fire(q):
            vec = ivm[pl.ds(r0, 1), pl.ds(c0 + q * 16, 16)][0]
            va = vec >> 3
            vc = vec & 7
            rbase = (i * _LK_PER_IT + q * 16) // 8
            cps = []
            for l in range(16):
                cps.append(pltpu.async_copy(
                    table_hbm.at[va[l], vc[l]],
                    rows_v.at[rbase + l // 8,
                              pl.ds((l % 8) * _EMBED_DIM, _EMBED_DIM)],
                    sem))
            return cps

        pend = {0: fire(0), 1: fire(1)}
        for q in range(_GRP):
            if q + 2 < _GRP:
                pend[q + 2] = fire(q + 2)
            for cp in pend.pop(q):
                cp.wait()
        return carry

    lax.fori_loop(0, _N_IT, step, 0)
    pltpu.sync_copy(rows_v, out_hbm.at[wid])


@functools.cache
def _get_sc_gather():
    return functools.partial(
        pl.kernel,
        out_type=jax.ShapeDtypeStruct(
            (_NW, _ROWS_PER_W // 8, 8 * _EMBED_DIM), jnp.float32),
        mesh=plsc.VectorSubcoreMesh(core_axis_name="c", subcore_axis_name="s"),
        scratch_types=[
            pltpu.VMEM((_CHUNKS_PER_W, _IDX_MINOR), jnp.int32),
            pltpu.VMEM((_ROWS_PER_W // 8, 8 * _EMBED_DIM), jnp.float32),
            pltpu.SemaphoreType.DMA,
        ],
    )(_sc_gather_body)


_BT = 256


def _tc_body(emb_ref, w0a_ref, w0b_ref, s0_ref, c0_ref, w1_ref, s1_ref,
             c1_ref, w2_ref, s2_ref, c2_ref, wo_ref, bo_ref, out_ref):
    a = emb_ref[...]
    et = jnp.swapaxes(a, 0, 1)

    ki = lax.broadcasted_iota(jnp.int32, (32, _NUM_FIELDS * _EMBED_DIM), 1)
    ri = lax.broadcasted_iota(jnp.int32, (32, _NUM_FIELDS * _EMBED_DIM), 0)
    r_mat = (ki // _EMBED_DIM == ri).astype(jnp.float32)

    blocks = []
    for j in range(1, _NUM_FIELDS):
        t = et[j * _EMBED_DIM:(j + 1) * _EMBED_DIM, :]
        tiled = jnp.concatenate([t] * _NUM_FIELDS, axis=0)
        prod = et * tiled
        blocks.append(jnp.dot(r_mat, prod,
                              preferred_element_type=jnp.float32))
    s_t = jnp.concatenate(blocks, axis=0)

    dn = (((0,), (0,)), ((), ()))
    z = (lax.dot_general(et, w0a_ref[...], dn,
                         preferred_element_type=jnp.float32)
         + lax.dot_general(s_t, w0b_ref[...], dn,
                           preferred_element_type=jnp.float32))
    z = jnp.maximum(z * s0_ref[...] + c0_ref[...], 0.0)
    z = jnp.dot(z, w1_ref[...], preferred_element_type=jnp.float32)
    z = jnp.maximum(z * s1_ref[...] + c1_ref[...], 0.0)
    z = jnp.dot(z, w2_ref[...], preferred_element_type=jnp.float32)
    z = jnp.maximum(z * s2_ref[...] + c2_ref[...], 0.0)
    out_ref[...] = (jnp.dot(z, wo_ref[...],
                            preferred_element_type=jnp.float32)
                    + bo_ref[...])


def _tc_mlp(emb, w0a, w0b, s0, c0, w1, s1, c1, w2, s2, c2, wo, bo):
    full = lambda shp: pl.BlockSpec(shp, lambda i: (0, 0))
    grid = _BATCH // _BT
    return pl.pallas_call(
        _tc_body,
        grid=(grid,),
        in_specs=[
            pl.BlockSpec((_BT, _NUM_FIELDS * _EMBED_DIM), lambda i: (i, 0)),
            full(w0a.shape), full(w0b.shape), full(s0.shape), full(c0.shape),
            full(w1.shape), full(s1.shape), full(c1.shape),
            full(w2.shape), full(s2.shape), full(c2.shape),
            full(wo.shape), full(bo.shape),
        ],
        out_specs=pl.BlockSpec((_BT, 1), lambda i: (i, 0)),
        out_shape=jax.ShapeDtypeStruct((_BATCH, 1), jnp.float32),
        compiler_params=pltpu.CompilerParams(
            dimension_semantics=("arbitrary",)),
    )(emb, w0a, w0b, s0, c0, w1, s1, c1, w2, s2, c2, wo, bo)


def kernel(x, params):
    idx = (x + jnp.asarray(_OFFSETS)[None, :]).reshape(
        _NW, _CHUNKS_PER_W, _IDX_MINOR)
    table3 = params['table'].reshape(-1, 8, _EMBED_DIM)
    rows = _get_sc_gather()(table3, idx)
    emb = rows.reshape(_BATCH, _NUM_FIELDS * _EMBED_DIM)

    inv = 1.0 / np.sqrt(1.0 + 1e-5).astype(np.float32)
    w0 = params['W0']
    w0a = w0[:_NUM_FIELDS * _EMBED_DIM]
    w0b = jnp.take(w0, jnp.asarray(_w0b_src), axis=0) * jnp.asarray(_w0b_valid)
    scs = []
    for i in range(3):
        s = (params[f'g{i}'] * inv)[None, :]
        c = (params[f'b{i}'] * s[0] + params[f'beta{i}'])[None, :]
        scs += [s, c]
    return _tc_mlp(emb, w0a, w0b, scs[0], scs[1], params['W1'], scs[2],
                   scs[3], params['W2'], scs[4], scs[5], params['Wo'],
                   params['bo'].reshape(1, 1))
